# baseline (speedup 1.0000x reference)
import functools

import jax
import jax.numpy as jnp
from jax import lax
from jax.experimental import pallas as pl
from jax.experimental.pallas import tpu as pltpu
from jax.experimental.pallas import tpu_sc as plsc

B = 4
S = 4096
HIDDEN = 1024

NC = 2
NS = 16
NW = NC * NS

TOTAL = B * S
B_PER_W = TOTAL // NW
W_PER_ROW = S // B_PER_W
CHUNK = 16
N_CHUNKS = B_PER_W // CHUNK
NBUF = 6


def _gather_body(idx_hbm, table_hbm, out_hbm, idx_v, rows_v, *sems):
    gsems, ssems = sems[:NBUF], sems[NBUF:]
    wid = lax.axis_index("s") * NC + lax.axis_index("c")
    row = wid // W_PER_ROW
    col = (wid % W_PER_ROW) * B_PER_W
    pltpu.sync_copy(idx_hbm.at[row, pl.ds(col, B_PER_W)], idx_v)

    def gather(i):
        return pltpu.async_copy(
            table_hbm.at[idx_v.at[pl.ds(i * CHUNK, CHUNK)]],
            rows_v.at[i % NBUF],
            gsems[i % NBUF],
        )

    def scatter(i):
        return pltpu.async_copy(
            rows_v.at[i % NBUF],
            out_hbm.at[row, pl.ds(col + i * CHUNK, CHUNK)],
            ssems[i % NBUF],
        )

    g = [None] * NBUF
    s = [None] * NBUF
    for i in range(NBUF - 1):
        g[i] = gather(i)
    for i in range(N_CHUNKS):
        b = i % NBUF
        nxt = i + NBUF - 1
        if nxt < N_CHUNKS:
            bn = nxt % NBUF
            if s[bn] is not None:
                s[bn].wait()
            g[bn] = gather(nxt)
        g[b].wait()
        s[b] = scatter(i)
    for i in range(max(0, N_CHUNKS - NBUF), N_CHUNKS):
        s[i % NBUF].wait()


@jax.jit
def _embed(x, weight):
    mesh = plsc.VectorSubcoreMesh(core_axis_name="c", subcore_axis_name="s")
    run = functools.partial(
        pl.kernel,
        mesh=mesh,
        out_type=jax.ShapeDtypeStruct((B, S, HIDDEN), jnp.float32),
        scratch_types=[
            pltpu.VMEM((B_PER_W,), jnp.int32),
            pltpu.VMEM((NBUF, CHUNK, HIDDEN), jnp.float32),
        ] + [pltpu.SemaphoreType.DMA] * (2 * NBUF),
    )(_gather_body)
    return run(x, weight)


def kernel(x, weight):
    return _embed(x, weight)


# speedup vs baseline: 1.0063x; 1.0063x over previous
import functools

import jax
import jax.numpy as jnp
from jax import lax
from jax.experimental import pallas as pl
from jax.experimental.pallas import tpu as pltpu
from jax.experimental.pallas import tpu_sc as plsc

B = 4
S = 4096
HIDDEN = 1024

NC = 2
NS = 16
NW = NC * NS

TOTAL = B * S
B_PER_W = TOTAL // NW
W_PER_ROW = S // B_PER_W
CHUNK = 16
N_CHUNKS = B_PER_W // CHUNK
NBUF = 6


def _gather_body(idx_hbm, table_hbm, out_hbm, idx_v, rows_v, *sems):
    gsems, ssems, isem = sems[:NBUF], sems[NBUF : 2 * NBUF], sems[2 * NBUF]
    wid = lax.axis_index("s") * NC + lax.axis_index("c")
    row = wid // W_PER_ROW
    col = (wid % W_PER_ROW) * B_PER_W
    head = 128
    pltpu.sync_copy(idx_hbm.at[row, pl.ds(col, head)], idx_v.at[pl.ds(0, head)])
    idx_rest = pltpu.async_copy(
        idx_hbm.at[row, pl.ds(col + head, B_PER_W - head)],
        idx_v.at[pl.ds(head, B_PER_W - head)],
        isem,
    )

    def gather(i):
        return pltpu.async_copy(
            table_hbm.at[idx_v.at[pl.ds(i * CHUNK, CHUNK)]],
            rows_v.at[i % NBUF],
            gsems[i % NBUF],
        )

    def scatter(i):
        return pltpu.async_copy(
            rows_v.at[i % NBUF],
            out_hbm.at[row, pl.ds(col + i * CHUNK, CHUNK)],
            ssems[i % NBUF],
        )

    g = [None] * NBUF
    s = [None] * NBUF
    for i in range(NBUF - 1):
        g[i] = gather(i)
    idx_rest.wait()
    for i in range(N_CHUNKS):
        b = i % NBUF
        nxt = i + NBUF - 1
        if nxt < N_CHUNKS:
            bn = nxt % NBUF
            if s[bn] is not None:
                s[bn].wait()
            g[bn] = gather(nxt)
        g[b].wait()
        s[b] = scatter(i)
    for i in range(max(0, N_CHUNKS - NBUF), N_CHUNKS):
        s[i % NBUF].wait()


@jax.jit
def _embed(x, weight):
    mesh = plsc.VectorSubcoreMesh(core_axis_name="c", subcore_axis_name="s")
    run = functools.partial(
        pl.kernel,
        mesh=mesh,
        out_type=jax.ShapeDtypeStruct((B, S, HIDDEN), jnp.float32),
        scratch_types=[
            pltpu.VMEM((B_PER_W,), jnp.int32),
            pltpu.VMEM((NBUF, CHUNK, HIDDEN), jnp.float32),
        ] + [pltpu.SemaphoreType.DMA] * (2 * NBUF + 1),
    )(_gather_body)
    return run(x, weight)


def kernel(x, weight):
    return _embed(x, weight)
